# baseline (speedup 1.0000x reference)
import functools

import jax
import jax.numpy as jnp
from jax import lax
from jax.experimental import pallas as pl
from jax.experimental.pallas import tpu as pltpu
from jax.experimental.pallas import tpu_sc as plsc

_B = 16384
_EMB = 64
_NCLS = 100000
_MARGIN = 0.01

_NC = 2
_NS = 16
_NW = _NC * _NS
_BPW = _B // _NW
_CH = 16
_NCHUNK = _BPW // _CH
_NTBL = 13


def _sqrtv(x):
    i = plsc.bitcast(x, jnp.int32)
    i = jnp.int32(0x5F3759DF) - lax.shift_right_arithmetic(i, 1)
    y = plsc.bitcast(i, jnp.float32)
    h = 0.5 * x
    y = y * (1.5 - h * y * y)
    y = y * (1.5 - h * y * y)
    y = y * (1.5 - h * y * y)
    return jnp.where(x > 0.0, x * y, 0.0)


def _relu(v):
    return jnp.maximum(v, 0.0)


def _reg(s):
    return jnp.abs(s - 1.0)


@functools.partial(
    pl.kernel,
    out_type=jax.ShapeDtypeStruct((_B,), jnp.float32),
    mesh=plsc.VectorSubcoreMesh(core_axis_name="c", subcore_axis_name="s"),
    compiler_params=pltpu.CompilerParams(
        use_tc_tiling_on_sc=False, needs_layout_passes=False),
    scratch_types=[
        pltpu.VMEM((2, _NTBL, _CH), jnp.int32),
        pltpu.VMEM((2, 11, 5 * _CH), jnp.int32),
    ] + [pltpu.VMEM((5 * _CH, 16), jnp.float32) for _ in range(22)]
      + [pltpu.VMEM((_CH, _EMB), jnp.float32) for _ in range(4)]
      + [
        pltpu.VMEM((_BPW,), jnp.float32),
        pltpu.SemaphoreType.DMA,
        pltpu.SemaphoreType.DMA,
    ],
)
def _sc_loss(idx_hbm, clsb_hbm, rel_hbm, out_hbm, idx_v, qidx, *rest):
    cbufs = (rest[0:11], rest[11:22])
    rbufs = (rest[22:24], rest[24:26])
    obuf = rest[26]
    sem = rest[27]
    semi = rest[28]
    wid = lax.axis_index("s") * _NC + lax.axis_index("c")
    base_cid = wid * _NCHUNK
    iota = lax.iota(jnp.int32, 16)

    def fire_gathers(t, s):
        for k in range(11):
            v = idx_v[s, k, pl.ds(0, 16)]
            q = lax.shift_right_logical(v * 65, 4)
            for m in range(5):
                plsc.store_scatter(qidx.at[s, k], [iota * 5 + m], q + m)
        for k in range(11):
            pltpu.async_copy(clsb_hbm.at[qidx.at[s, k]], cbufs[s][k], sem)
        for k in range(2):
            pltpu.async_copy(rel_hbm.at[idx_v.at[s, 11 + k]], rbufs[s][k], sem)

    def wait_gathers(s):
        for k in range(11):
            pltpu.make_async_copy(
                clsb_hbm.at[pl.ds(0, 5 * _CH)], cbufs[s][k], sem).wait()
        for k in range(2):
            pltpu.make_async_copy(
                rel_hbm.at[pl.ds(0, _CH)], rbufs[s][k], sem).wait()

    def issue_idx(t, s):
        pltpu.async_copy(idx_hbm.at[base_cid + t], idx_v.at[s], semi)

    def wait_idx(s):
        pltpu.make_async_copy(idx_hbm.at[0], idx_v.at[s], semi).wait()

    def compute(t, s):
        row5 = iota * 5
        row5r = row5 + 4
        off = [jnp.bitwise_and(idx_v[s, k, pl.ds(0, 16)], 15)
               for k in range(11)]

        def cld(k, u):
            return plsc.load_gather(
                cbufs[s][k],
                [row5 + lax.shift_right_logical(u, 4),
                 jnp.bitwise_and(u, 15)])

        def col(j, acc):
            (a1cd, a1c, a1d,
             a2dc, a2ec, a2ed, a2c, a2d, a2e,
             a3td, a3t, a3d,
             a4dt, a4t, a4d,
             a5cd, a5c, a5d) = acc
            cj = jnp.full((16,), j, jnp.int32)

            c = cld(0, off[0] + cj)
            d = cld(1, off[1] + cj)
            t1 = c - d
            a1cd += t1 * t1
            a1c += c * c
            a1d += d * d

            c = cld(2, off[2] + cj)
            d = cld(3, off[3] + cj)
            e = cld(4, off[4] + cj)
            t1 = d - c
            a2dc += t1 * t1
            t1 = e - c
            a2ec += t1 * t1
            t1 = e - d
            a2ed += t1 * t1
            a2c += c * c
            a2d += d * d
            a2e += e * e

            c = cld(5, off[5] + cj)
            d = cld(6, off[6] + cj)
            r = plsc.load_gather(rbufs[s][0], [iota, cj])
            tt = c + r
            u = tt - d
            a3td += u * u
            a3t += tt * tt
            a3d += d * d

            c = cld(7, off[7] + cj)
            d = cld(8, off[8] + cj)
            r = plsc.load_gather(rbufs[s][1], [iota, cj])
            tt = c - r
            u = d - tt
            a4dt += u * u
            a4t += tt * tt
            a4d += d * d

            c = cld(9, off[9] + cj)
            d = cld(10, off[10] + cj)
            t1 = c - d
            a5cd += t1 * t1
            a5c += c * c
            a5d += d * d

            return (a1cd, a1c, a1d,
                    a2dc, a2ec, a2ed, a2c, a2d, a2e,
                    a3td, a3t, a3d,
                    a4dt, a4t, a4d,
                    a5cd, a5c, a5d)

        z = jnp.zeros((16,), jnp.float32)
        (a1cd, a1c, a1d,
         a2dc, a2ec, a2ed, a2c, a2d, a2e,
         a3td, a3t, a3d,
         a4dt, a4t, a4d,
         a5cd, a5c, a5d) = lax.fori_loop(0, _EMB, col, (z,) * 18)

        rad = [jnp.abs(plsc.load_gather(cbufs[s][k], [row5r, off[k]]))
               for k in range(11)]

        l1 = (_relu(_sqrtv(a1cd) + rad[0] - rad[1])
              + _reg(_sqrtv(a1c)) + _reg(_sqrtv(a1d)))
        rc2, rd2 = rad[2], rad[3]
        l2 = (_relu(_sqrtv(a2dc) - (rc2 + rd2))
              + _relu(_sqrtv(a2ec) - rc2)
              + _relu(_sqrtv(a2ed) - rd2)
              - _MARGIN
              + _reg(_sqrtv(a2c)) + _reg(_sqrtv(a2d)) + _reg(_sqrtv(a2e)))
        l3 = (_relu(_sqrtv(a3td) + rad[5] - rad[6])
              + _reg(_sqrtv(a3t)) + _reg(_sqrtv(a3d)))
        l4 = (_relu(_sqrtv(a4dt) - rad[7] - rad[8] - _MARGIN)
              + _reg(_sqrtv(a4t)) + _reg(_sqrtv(a4d)))
        l5 = (_relu(rad[9] + rad[10] - _sqrtv(a5cd) + _MARGIN)
              + _reg(_sqrtv(a5c)) + _reg(_sqrtv(a5d)))

        obuf[pl.ds(t * _CH, 16)] = l1 + l2 + l3 + l4 + l5

    issue_idx(0, 0)
    wait_idx(0)
    fire_gathers(0, 0)
    issue_idx(1, 1)

    def pbody(p, carry):
        for s in range(2):
            t = 2 * p + s
            wait_gathers(s)

            @pl.when(t + 1 < _NCHUNK)
            def _():
                wait_idx(1 - s)
                fire_gathers(t + 1, 1 - s)

            @pl.when(t + 2 < _NCHUNK)
            def _():
                issue_idx(t + 2, s)

            compute(t, s)
        return carry

    lax.fori_loop(0, _NCHUNK // 2, pbody, 0)
    pltpu.sync_copy(obuf, out_hbm.at[pl.ds(wid * _BPW, _BPW)])


def kernel(nf1, nf2, nf3, nf4, dis, cls_emb, rel_emb):
    cols = [nf1[:, 0], nf1[:, 1],
            nf2[:, 0], nf2[:, 1], nf2[:, 2],
            nf3[:, 0], nf3[:, 2],
            nf4[:, 1], nf4[:, 2],
            dis[:, 0], dis[:, 1],
            nf3[:, 1], nf4[:, 0]]
    idx = jnp.stack(
        [c.astype(jnp.int32).reshape(_B // _CH, _CH) for c in cols], axis=1)
    clsb = cls_emb.reshape(_NCLS * (_EMB + 1) // 16, 16)
    out = _sc_loss(idx, clsb, rel_emb)
    return out.reshape(_B, 1)
